# baseline (speedup 1.0000x reference)
import functools

import jax
import jax.numpy as jnp
from jax.experimental import pallas as pl
from jax.experimental.pallas import tpu as pltpu


def _gather_rows(idx_ref, table_ref, logits_ref, base, R, unroll=8):
    n_batches = R // unroll

    def load_batch(b):
        return tuple(table_ref[idx_ref[base + b * unroll + k], 0]
                     for k in range(unroll))

    def body(b, carry):
        new_vals = load_batch(b)
        o = (b - 1) * unroll
        for k in range(unroll):
            logits_ref[o + k, 0] = carry[k]
        return new_vals

    last = jax.lax.fori_loop(1, n_batches, body, load_batch(0))
    o = (n_batches - 1) * unroll
    for k in range(unroll):
        logits_ref[o + k, 0] = last[k]


def _logits_kernel(idx_ref, table_ref, logits_ref):
    i = pl.program_id(0)
    R = logits_ref.shape[0]
    _gather_rows(idx_ref, table_ref, logits_ref, i * R, R)


def _fold(x, op):
    Vp = x.shape[2]
    a = x[:, :, :1024]
    for g in range(1, Vp // 1024):
        a = op(a, x[:, :, g * 1024:(g + 1) * 1024])
    w = 1024
    while w > 128:
        w //= 2
        a = op(a[:, :, :w], a[:, :, w:])
    return a


def _loss_kernel(idx_ref, table_ref, tgt_ref, logits_ref, rowloss_ref,
                 scratch_ref, *, bt):
    i = pl.program_id(0)
    R, _, V = logits_ref.shape
    Vp = scratch_ref.shape[2]
    base = i * R
    unroll = 8

    scratch_ref[:, :, V:] = jnp.full((R, 1, Vp - V), -jnp.inf, jnp.float32)

    n_batches = R // unroll

    def load_batch(b):
        return tuple(table_ref[idx_ref[base + b * unroll + k], 0]
                     for k in range(unroll))

    def body(b, carry):
        new_vals = load_batch(b)
        o = (b - 1) * unroll
        for k in range(unroll):
            logits_ref[o + k, 0] = carry[k]
            scratch_ref[o + k, 0, :V] = carry[k]
        return new_vals

    last = jax.lax.fori_loop(1, n_batches, body, load_batch(0))
    o = (n_batches - 1) * unroll
    for k in range(unroll):
        logits_ref[o + k, 0] = last[k]
        scratch_ref[o + k, 0, :V] = last[k]

    x = scratch_ref[...]

    m = jnp.max(_fold(x, jnp.maximum), axis=2, keepdims=True)
    e = jnp.exp(x - m)
    s = jnp.sum(_fold(e, jnp.add), axis=2, keepdims=True)
    lse = jnp.log(s) + m

    tgt = tgt_ref[...]
    col = jax.lax.broadcasted_iota(jnp.int32, (R, 1, Vp), 2)
    masked = jnp.where(col == tgt, x, 0.0)
    tl = jnp.sum(_fold(masked, jnp.add), axis=2, keepdims=True)

    loss = lse - tl
    if bt is not None:
        row_ids = base + jax.lax.broadcasted_iota(jnp.int32, (R, 1, 1), 0)
        loss = jnp.where(row_ids < bt, loss, 0.0)
    rowloss_ref[...] = loss


def _chunking(bt):
    r = 512 if bt >= 512 else ((bt + 7) // 8) * 8
    bt_pad = ((bt + r - 1) // r) * r
    return r, bt_pad


def _vmem_limit(v, r):
    table_b = v * v * 4
    blocks_b = 4 * r * v * 4 + 2 * r * 3072 * 4 + (2 << 20)
    return int(min(table_b + blocks_b + (6 << 20), 60 << 20))


def _forward(idx, targets, table):
    B, T = idx.shape
    V = table.shape[0]
    BT = B * T
    R, BT_pad = _chunking(BT)
    num_chunks = BT_pad // R

    idx_flat = idx.reshape(BT).astype(jnp.int32)
    idx_pad = jnp.pad(idx_flat, (0, BT_pad - BT))
    table3 = table.reshape(V, 1, V)

    compiler_params = pltpu.CompilerParams(
        dimension_semantics=("parallel",),
        vmem_limit_bytes=_vmem_limit(V, R),
    )

    if targets is None:
        logits = pl.pallas_call(
            _logits_kernel,
            out_shape=jax.ShapeDtypeStruct((BT_pad, 1, V), table.dtype),
            grid_spec=pltpu.PrefetchScalarGridSpec(
                num_scalar_prefetch=1,
                grid=(num_chunks,),
                in_specs=[
                    pl.BlockSpec((V, 1, V), lambda i, idx_ref: (0, 0, 0)),
                ],
                out_specs=pl.BlockSpec((R, 1, V), lambda i, idx_ref: (i, 0, 0)),
            ),
            compiler_params=compiler_params,
        )(idx_pad, table3)
        return logits[:BT].reshape(B, T, V), None

    tgt_flat = targets.reshape(BT).astype(jnp.int32)
    tgt_pad = jnp.pad(tgt_flat, (0, BT_pad - BT)).reshape(BT_pad, 1, 1)

    kern = functools.partial(_loss_kernel, bt=None if BT_pad == BT else BT)
    Vp = ((V + 1023) // 1024) * 1024

    logits, rowloss = pl.pallas_call(
        kern,
        out_shape=(
            jax.ShapeDtypeStruct((BT_pad, 1, V), table.dtype),
            jax.ShapeDtypeStruct((BT_pad, 1, 1), jnp.float32),
        ),
        grid_spec=pltpu.PrefetchScalarGridSpec(
            num_scalar_prefetch=1,
            grid=(num_chunks,),
            in_specs=[
                pl.BlockSpec((V, 1, V), lambda i, idx_ref: (0, 0, 0)),
                pl.BlockSpec((R, 1, 1), lambda i, idx_ref: (i, 0, 0)),
            ],
            out_specs=(
                pl.BlockSpec((R, 1, V), lambda i, idx_ref: (i, 0, 0)),
                pl.BlockSpec((R, 1, 1), lambda i, idx_ref: (i, 0, 0)),
            ),
            scratch_shapes=[pltpu.VMEM((R, 1, Vp), jnp.float32)],
        ),
        compiler_params=compiler_params,
    )(idx_pad, table3, tgt_pad)

    loss = jnp.sum(rowloss) / BT
    return logits[:BT].reshape(BT, V), loss


def kernel(idx, targets, table):
    return _forward(idx, targets, table)


# speedup vs baseline: 1.6980x; 1.6980x over previous
import functools

import jax
import jax.numpy as jnp
from jax.experimental import pallas as pl
from jax.experimental.pallas import tpu as pltpu


def _gather_kernel(idx_ref, table_ref, logits_ref, *, unroll=8):
    i = pl.program_id(0)
    R = logits_ref.shape[0]
    base = i * R
    n_batches = R // unroll

    def load_batch(b):
        return tuple(table_ref[idx_ref[base + b * unroll + k], 0]
                     for k in range(unroll))

    def body(b, carry):
        new_vals = load_batch(b)
        o = (b - 1) * unroll
        for k in range(unroll):
            logits_ref[o + k, 0] = carry[k]
        return new_vals

    last = jax.lax.fori_loop(1, n_batches, body, load_batch(0))
    o = (n_batches - 1) * unroll
    for k in range(unroll):
        logits_ref[o + k, 0] = last[k]


def _ce_kernel(logits_ref, tgt_ref, rowloss_ref, *, bt):
    i = pl.program_id(0)
    R, V = logits_ref.shape
    rows = logits_ref[...]

    m = jnp.max(rows, axis=-1, keepdims=True)
    s = jnp.sum(jnp.exp(rows - m), axis=-1, keepdims=True)
    lse = jnp.log(s) + m

    tgt = tgt_ref[...]
    col = jax.lax.broadcasted_iota(jnp.int32, (R, V), 1)
    tl = jnp.sum(jnp.where(col == tgt, rows, 0.0),
                 axis=-1, keepdims=True)

    loss = lse - tl
    if bt is not None:
        row_ids = i * R + jax.lax.broadcasted_iota(jnp.int32, (R, 1), 0)
        loss = jnp.where(row_ids < bt, loss, 0.0)
    rowloss_ref[...] = loss


def _chunking(bt):
    r = 1024 if bt >= 1024 else ((bt + 7) // 8) * 8
    bt_pad = ((bt + r - 1) // r) * r
    return r, bt_pad


def _gather(idx_pad, table3, BT_pad, R, V, dtype):
    num_chunks = BT_pad // R
    return pl.pallas_call(
        _gather_kernel,
        out_shape=jax.ShapeDtypeStruct((BT_pad, 1, V), dtype),
        grid_spec=pltpu.PrefetchScalarGridSpec(
            num_scalar_prefetch=1,
            grid=(num_chunks,),
            in_specs=[
                pl.BlockSpec((V, 1, V), lambda i, idx_ref: (0, 0, 0)),
            ],
            out_specs=pl.BlockSpec((R, 1, V), lambda i, idx_ref: (i, 0, 0)),
        ),
        compiler_params=pltpu.CompilerParams(
            dimension_semantics=("parallel",),
            vmem_limit_bytes=int(min(
                V * V * 4 + 5 * R * V * 4 + (6 << 20), 60 << 20)),
        ),
    )(idx_pad, table3)


def _forward(idx, targets, table):
    B, T = idx.shape
    V = table.shape[0]
    BT = B * T
    R, BT_pad = _chunking(BT)
    num_chunks = BT_pad // R

    idx_flat = idx.reshape(BT).astype(jnp.int32)
    idx_pad = jnp.pad(idx_flat, (0, BT_pad - BT))
    table3 = table.reshape(V, 1, V)

    logits3 = _gather(idx_pad, table3, BT_pad, R, V, table.dtype)

    if targets is None:
        return logits3[:BT].reshape(B, T, V), None

    logits2 = logits3.reshape(BT_pad, V)
    tgt_flat = targets.reshape(BT).astype(jnp.int32)
    tgt_pad = jnp.pad(tgt_flat, (0, BT_pad - BT)).reshape(BT_pad, 1)

    ce = functools.partial(_ce_kernel, bt=None if BT_pad == BT else BT)
    rowloss = pl.pallas_call(
        ce,
        out_shape=jax.ShapeDtypeStruct((BT_pad, 1), jnp.float32),
        grid=(num_chunks,),
        in_specs=[
            pl.BlockSpec((R, V), lambda i: (i, 0)),
            pl.BlockSpec((R, 1), lambda i: (i, 0)),
        ],
        out_specs=pl.BlockSpec((R, 1), lambda i: (i, 0)),
        compiler_params=pltpu.CompilerParams(
            dimension_semantics=("parallel",),
            vmem_limit_bytes=int(min(4 * R * V * 4 + (4 << 20), 60 << 20)),
        ),
    )(logits2, tgt_pad)

    loss = jnp.sum(rowloss) / BT
    return logits2[:BT].reshape(BT, V), loss


def kernel(idx, targets, table):
    return _forward(idx, targets, table)
